# baseline (speedup 1.0000x reference)
import jax
import jax.numpy as jnp
from jax.experimental import pallas as pl
from jax.experimental.pallas import tpu as pltpu

B, N, K = 128, 32768, 10
RPB = 64
STEPS = B // RPB
NSEG = K * RPB
W = 512
NEG = float('-inf')


def _issue_patches(seg_ref, zout_ref, t_smem, psem, step, buf):
    pcopies = []
    for r in range(RPB):
        iscal = [t_smem[buf, k * RPB + r] for k in range(K)]
        for k in range(K):
            rank = (iscal[0] < iscal[k]).astype(jnp.int32) if k else 0
            for m in range(1, K):
                if m != k:
                    rank = rank + (iscal[m] < iscal[k]).astype(jnp.int32)
            c = pltpu.make_async_copy(
                seg_ref.at[buf, k * RPB + r],
                zout_ref.at[rank * B + step * RPB + r,
                            pl.ds((iscal[k] // W) * W, W)],
                psem.at[buf])
            c.start()
            pcopies.append(c)
    return pcopies


def _body(x_ref, zout_ref, zs_ref, s_ref, seg_ref, t_smem, zsem, psem):
    step = pl.program_id(0)
    buf = jax.lax.rem(step, 2)

    @pl.when(step == 0)
    def _():
        zs_ref[...] = jnp.zeros_like(zs_ref)

    zcopies = [
        pltpu.make_async_copy(
            zs_ref, zout_ref.at[pl.ds(j * B + step * RPB, RPB), :], zsem)
        for j in range(K)
    ]
    for c in zcopies:
        c.start()

    @pl.when(step > 0)
    def _():
        _issue_patches(seg_ref, zout_ref, t_smem, psem, step - 1, 1 - buf)

    @pl.when(step > 1)
    def _():
        for _ in range(NSEG):
            pltpu.make_async_copy(
                seg_ref.at[0, 0], zout_ref.at[0, pl.ds(0, W)],
                psem.at[buf]).wait()

    li = jax.lax.broadcasted_iota(jnp.int32, (RPB, N), 1)
    s_ref[...] = x_ref[...]
    idxs = []
    for _ in range(K):
        s = s_ref[...]
        v = jnp.max(s, axis=1, keepdims=True)
        i = jnp.min(jnp.where(s == v, li, N), axis=1, keepdims=True)
        idxs.append(i)
        s_ref[...] = jnp.where(li == i, NEG, s)

    ci = jax.lax.broadcasted_iota(jnp.int32, (RPB, W), 1)
    for k in range(K):
        seg_ref[buf, k * RPB:(k + 1) * RPB, :] = (
            ci == idxs[k] % W).astype(jnp.float32)

    ri = jax.lax.broadcasted_iota(jnp.int32, (RPB, 1), 0)
    for k in range(K):
        for r in range(RPB):
            t_smem[buf, k * RPB + r] = jnp.sum(jnp.where(ri == r, idxs[k], 0))

    for c in zcopies:
        c.wait()

    @pl.when(step == STEPS - 1)
    def _():
        pc = _issue_patches(seg_ref, zout_ref, t_smem, psem, step, buf)
        for c in pc:
            c.wait()
        for _ in range(NSEG):
            pltpu.make_async_copy(
                seg_ref.at[0, 0], zout_ref.at[0, pl.ds(0, W)],
                psem.at[1 - buf]).wait()


def kernel(x):
    planes = pl.pallas_call(
        _body,
        grid=(STEPS,),
        in_specs=[pl.BlockSpec((RPB, N), lambda i: (i, 0))],
        out_specs=pl.BlockSpec(memory_space=pl.ANY),
        out_shape=jax.ShapeDtypeStruct((K * B, N), jnp.float32),
        scratch_shapes=[
            pltpu.VMEM((RPB, N), jnp.float32),
            pltpu.VMEM((RPB, N), jnp.float32),
            pltpu.VMEM((2, NSEG, W), jnp.float32),
            pltpu.SMEM((2, NSEG), jnp.int32),
            pltpu.SemaphoreType.DMA,
            pltpu.SemaphoreType.DMA((2,)),
        ],
    )(x)
    return jnp.transpose(planes.reshape(K, B, N), (1, 2, 0))


# speedup vs baseline: 1.5881x; 1.5881x over previous
import jax
import jax.numpy as jnp
from jax.experimental import pallas as pl
from jax.experimental.pallas import tpu as pltpu

B, N, K = 128, 32768, 10
RPB = 32
STEPS = B // RPB
NSEG = K * RPB
W = 512
NEG = float('-inf')


def _issue_patches(seg_ref, zout_ref, t_smem, psem, step, buf):
    pcopies = []
    for r in range(RPB):
        iscal = [t_smem[buf, k * RPB + r] for k in range(K)]
        for k in range(K):
            rank = (iscal[0] < iscal[k]).astype(jnp.int32) if k else 0
            for m in range(1, K):
                if m != k:
                    rank = rank + (iscal[m] < iscal[k]).astype(jnp.int32)
            c = pltpu.make_async_copy(
                seg_ref.at[buf, k * RPB + r],
                zout_ref.at[rank * B + step * RPB + r,
                            pl.ds((iscal[k] // W) * W, W)],
                psem.at[buf])
            c.start()
            pcopies.append(c)
    return pcopies


def _body(x_ref, zout_ref, zs_ref, s_ref, seg_ref, t_smem, zsem, psem):
    step = pl.program_id(0)
    buf = jax.lax.rem(step, 2)

    @pl.when(step == 0)
    def _():
        zs_ref[...] = jnp.zeros_like(zs_ref)

    zcopies = [
        pltpu.make_async_copy(
            zs_ref, zout_ref.at[pl.ds(j * B + step * RPB, RPB), :], zsem)
        for j in range(K)
    ]
    for c in zcopies:
        c.start()

    @pl.when(step > 0)
    def _():
        _issue_patches(seg_ref, zout_ref, t_smem, psem, step - 1, 1 - buf)

    @pl.when(step > 1)
    def _():
        for _ in range(NSEG):
            pltpu.make_async_copy(
                seg_ref.at[0, 0], zout_ref.at[0, pl.ds(0, W)],
                psem.at[buf]).wait()

    li = jax.lax.broadcasted_iota(jnp.int32, (RPB, N), 1)
    s_ref[...] = x_ref[...]
    idxs = []
    for _ in range(K):
        s = s_ref[...]
        v = jnp.max(s, axis=1, keepdims=True)
        i = jnp.min(jnp.where(s == v, li, N), axis=1, keepdims=True)
        idxs.append(i)
        s_ref[...] = jnp.where(li == i, NEG, s)

    ci = jax.lax.broadcasted_iota(jnp.int32, (RPB, W), 1)
    for k in range(K):
        seg_ref[buf, k * RPB:(k + 1) * RPB, :] = (
            ci == idxs[k] % W).astype(jnp.float32)

    ri = jax.lax.broadcasted_iota(jnp.int32, (RPB, 1), 0)
    for k in range(K):
        for r in range(RPB):
            t_smem[buf, k * RPB + r] = jnp.sum(jnp.where(ri == r, idxs[k], 0))

    for c in zcopies:
        c.wait()

    @pl.when(step == STEPS - 1)
    def _():
        pc = _issue_patches(seg_ref, zout_ref, t_smem, psem, step, buf)
        for c in pc:
            c.wait()
        for _ in range(NSEG):
            pltpu.make_async_copy(
                seg_ref.at[0, 0], zout_ref.at[0, pl.ds(0, W)],
                psem.at[1 - buf]).wait()


def kernel(x):
    planes = pl.pallas_call(
        _body,
        grid=(STEPS,),
        in_specs=[pl.BlockSpec((RPB, N), lambda i: (i, 0))],
        out_specs=pl.BlockSpec(memory_space=pl.ANY),
        out_shape=jax.ShapeDtypeStruct((K * B, N), jnp.float32),
        scratch_shapes=[
            pltpu.VMEM((RPB, N), jnp.float32),
            pltpu.VMEM((RPB, N), jnp.float32),
            pltpu.VMEM((2, NSEG, W), jnp.float32),
            pltpu.SMEM((2, NSEG), jnp.int32),
            pltpu.SemaphoreType.DMA,
            pltpu.SemaphoreType.DMA((2,)),
        ],
    )(x)
    return jnp.transpose(planes.reshape(K, B, N), (1, 2, 0))
